# baseline (speedup 1.0000x reference)
import functools

import jax
import jax.numpy as jnp
from jax import lax
from jax.experimental import pallas as pl
from jax.experimental.pallas import tpu as pltpu
from jax.experimental.pallas import tpu_sc as plsc

B = 4096
L = 200
EMB = 64
NW = 32
BPW = B // NW
CHUNKS = ((0, 128), (128, 72))
QV = EMB // 16
WPAD = 2 * EMB


def _body(idx_hbm, w_hbm, out_hbm, idx_v, rows_v, out_v, sem0, sem1):
    c = lax.axis_index("c")
    s = lax.axis_index("s")
    wid = s * 2 + c
    base = wid * BPW * L

    pltpu.sync_copy(idx_hbm.at[pl.ds(base, BPW * L)], idx_v)

    sems = (sem0, sem1)

    def start(b, slot):
        for (o, n) in CHUNKS:
            pltpu.async_copy(
                w_hbm.at[idx_v.at[pl.ds(b * L + o, n)]],
                rows_v.at[slot, pl.ds(o, n)],
                sems[slot],
            )

    def wait(slot):
        for (o, n) in CHUNKS:
            pltpu.make_async_copy(
                w_hbm.at[idx_v.at[pl.ds(o, n)]],
                rows_v.at[slot, pl.ds(o, n)],
                sems[slot],
            ).wait()

    start(0, 0)
    start(1, 1)

    def accum(slot, b):
        def inner(l, acc):
            return tuple(
                acc[q] + rows_v[slot, l, pl.ds(16 * q, 16)] for q in range(QV)
            )
        zero = jnp.zeros((16,), jnp.float32)
        acc = lax.fori_loop(0, L, inner, (zero,) * QV)
        scale = jnp.float32(1.0 / L)
        for q in range(QV):
            out_v[b // 2, pl.ds((b % 2) * EMB + 16 * q, 16)] = acc[q] * scale

    def outer(g, carry):
        for slot in range(2):
            b = g * 2 + slot
            wait(slot)
            accum(slot, b)
            nb = b + 2

            @pl.when(nb < BPW)
            def _():
                start(nb, slot)
        return carry

    lax.fori_loop(0, BPW // 2, outer, 0)

    pltpu.sync_copy(out_v, out_hbm.at[pl.ds(wid * (BPW // 2), BPW // 2)])


_mesh = plsc.VectorSubcoreMesh(core_axis_name="c", subcore_axis_name="s")

_sc_call = pl.kernel(
    _body,
    mesh=_mesh,
    out_type=jax.ShapeDtypeStruct((B // 2, WPAD), jnp.float32),
    scratch_types=[
        pltpu.VMEM((BPW * L,), jnp.int32),
        pltpu.VMEM((2, L, WPAD), jnp.float32),
        pltpu.VMEM((BPW // 2, WPAD), jnp.float32),
        pltpu.SemaphoreType.DMA,
        pltpu.SemaphoreType.DMA,
    ],
    compiler_params=pltpu.CompilerParams(use_tc_tiling_on_sc=True),
)


@jax.jit
def _run(x, w):
    w128 = jnp.zeros((w.shape[0], WPAD), jnp.float32).at[:, :EMB].set(w)
    out2 = _sc_call(x.reshape(B * L), w128)
    return out2.reshape(B, EMB)


def kernel(x, sizes, emb_weight):
    del sizes
    return _run(x, emb_weight)


# speedup vs baseline: 1.4243x; 1.4243x over previous
import functools

import jax
import jax.numpy as jnp
from jax import lax
from jax.experimental import pallas as pl
from jax.experimental.pallas import tpu as pltpu
from jax.experimental.pallas import tpu_sc as plsc

B = 4096
L = 200
EMB = 64
NW = 32
BPW = B // NW
CHUNKS = ((0, 128), (128, 72))
QV = EMB // 16
WPAD = 2 * EMB


def _body(idx_hbm, w_hbm, out_hbm, idx_v, rows_v, out_v, sem0, sem1):
    c = lax.axis_index("c")
    s = lax.axis_index("s")
    wid = s * 2 + c
    base = wid * BPW * L

    pltpu.sync_copy(idx_hbm.at[pl.ds(base, BPW * L)], idx_v)

    sems = (sem0, sem1)

    def start(b, slot):
        for (o, n) in CHUNKS:
            pltpu.async_copy(
                w_hbm.at[idx_v.at[pl.ds(b * L + o, n)]],
                rows_v.at[slot, pl.ds(o, n)],
                sems[slot],
            )

    def wait(slot):
        for (o, n) in CHUNKS:
            pltpu.make_async_copy(
                w_hbm.at[idx_v.at[pl.ds(o, n)]],
                rows_v.at[slot, pl.ds(o, n)],
                sems[slot],
            ).wait()

    start(0, 0)
    start(1, 1)

    def accum(slot, b):
        def inner(l, acc):
            return tuple(
                acc[q] + rows_v[slot, l, pl.ds(16 * q, 16)] for q in range(QV)
            )
        zero = jnp.zeros((16,), jnp.float32)
        acc = lax.fori_loop(0, L, inner, (zero,) * QV)
        scale = jnp.float32(1.0 / L)
        for q in range(QV):
            out_v[b // 2, pl.ds((b % 2) * EMB + 16 * q, 16)] = acc[q] * scale

    def outer(g, carry):
        for slot in range(2):
            b = g * 2 + slot
            wait(slot)
            accum(slot, b)
            nb = b + 2

            @pl.when(nb < BPW)
            def _():
                start(nb, slot)
        return carry

    lax.fori_loop(0, BPW // 2, outer, 0)

    pltpu.sync_copy(out_v, out_hbm.at[pl.ds(wid * (BPW // 2), BPW // 2)])


_mesh = plsc.VectorSubcoreMesh(core_axis_name="c", subcore_axis_name="s")

_sc_call = pl.kernel(
    _body,
    mesh=_mesh,
    out_type=jax.ShapeDtypeStruct((B // 2, WPAD), jnp.float32),
    scratch_types=[
        pltpu.VMEM((BPW * L,), jnp.int32),
        pltpu.VMEM((2, L, WPAD), jnp.float32),
        pltpu.VMEM((BPW // 2, WPAD), jnp.float32),
        pltpu.SemaphoreType.DMA,
        pltpu.SemaphoreType.DMA,
    ],
    compiler_params=pltpu.CompilerParams(use_tc_tiling_on_sc=True),
)


@jax.jit
def _run(x, w):
    w128 = jnp.pad(w.T, ((0, WPAD - EMB), (0, 0))).T
    out2 = _sc_call(x.reshape(B * L), w128)
    return out2.reshape(B, EMB)


def kernel(x, sizes, emb_weight):
    del sizes
    return _run(x, emb_weight)


# speedup vs baseline: 1.5579x; 1.0938x over previous
import functools

import jax
import jax.numpy as jnp
from jax import lax
from jax.experimental import pallas as pl
from jax.experimental.pallas import tpu as pltpu
from jax.experimental.pallas import tpu_sc as plsc

B = 4096
L = 200
EMB = 64
NW = 32
BPW = B // NW
CHUNKS = ((0, 128), (128, 72))
QV = EMB // 16
WPAD = 2 * EMB


def _body(idx_hbm, w_hbm, out_hbm, idx_v, rows_v, out_v, sem0, sem1):
    c = lax.axis_index("c")
    s = lax.axis_index("s")
    wid = s * 2 + c
    base = wid * BPW * L

    pltpu.sync_copy(idx_hbm.at[pl.ds(base, BPW * L)], idx_v)

    sems = (sem0, sem1)

    def start(b, slot):
        for (o, n) in CHUNKS:
            pltpu.async_copy(
                w_hbm.at[idx_v.at[pl.ds(b * L + o, n)]],
                rows_v.at[slot, pl.ds(o, n)],
                sems[slot],
            )

    def wait(slot):
        for (o, n) in CHUNKS:
            pltpu.make_async_copy(
                w_hbm.at[idx_v.at[pl.ds(o, n)]],
                rows_v.at[slot, pl.ds(o, n)],
                sems[slot],
            ).wait()

    start(0, 0)
    start(1, 1)

    def accum(slot, b):
        def inner(l, acc):
            return tuple(
                acc[q] + rows_v[slot, l, pl.ds(16 * q, 16)] for q in range(QV)
            )
        zero = jnp.zeros((16,), jnp.float32)
        acc = lax.fori_loop(0, L, inner, (zero,) * QV)
        scale = jnp.float32(1.0 / L)
        for q in range(QV):
            out_v[b // 2, pl.ds((b % 2) * EMB + 16 * q, 16)] = acc[q] * scale

    def outer(g, carry):
        for slot in range(2):
            b = g * 2 + slot
            wait(slot)
            accum(slot, b)
            nb = b + 2

            @pl.when(nb < BPW)
            def _():
                start(nb, slot)
        return carry

    lax.fori_loop(0, BPW // 2, outer, 0)

    pltpu.sync_copy(out_v, out_hbm.at[pl.ds(wid * (BPW // 2), BPW // 2)])


BK = 2048


def _repack_body(in_ref, out_ref):
    out_ref[:, :EMB] = in_ref[...].T


def _repack(wt):
    n = wt.shape[1]
    grid = (n + BK - 1) // BK
    return pl.pallas_call(
        _repack_body,
        grid=(grid,),
        in_specs=[pl.BlockSpec((EMB, BK), lambda j: (0, j))],
        out_specs=pl.BlockSpec((BK, WPAD), lambda j: (j, 0)),
        out_shape=jax.ShapeDtypeStruct((n, WPAD), jnp.float32),
    )(wt)


_mesh = plsc.VectorSubcoreMesh(core_axis_name="c", subcore_axis_name="s")

_sc_call = pl.kernel(
    _body,
    mesh=_mesh,
    out_type=jax.ShapeDtypeStruct((B // 2, WPAD), jnp.float32),
    scratch_types=[
        pltpu.VMEM((BPW * L,), jnp.int32),
        pltpu.VMEM((2, L, WPAD), jnp.float32),
        pltpu.VMEM((BPW // 2, WPAD), jnp.float32),
        pltpu.SemaphoreType.DMA,
        pltpu.SemaphoreType.DMA,
    ],
    compiler_params=pltpu.CompilerParams(use_tc_tiling_on_sc=True),
)


@jax.jit
def _run(x, w):
    w128 = _repack(w.T)
    out2 = _sc_call(x.reshape(B * L), w128)
    return out2.reshape(B, EMB)


def kernel(x, sizes, emb_weight):
    del sizes
    return _run(x, emb_weight)


# speedup vs baseline: 1.8674x; 1.1986x over previous
import functools

import jax
import jax.numpy as jnp
from jax import lax
from jax.experimental import pallas as pl
from jax.experimental.pallas import tpu as pltpu
from jax.experimental.pallas import tpu_sc as plsc

B = 4096
L = 200
EMB = 64
NW = 32
BPW = B // NW
CHUNKS = ((0, 128), (128, 72))
QV = EMB // 16
WPAD = 2 * EMB


def _body(idx_hbm, w_hbm, out_hbm, idx_v, rows_v, out_v, sem0, sem1):
    c = lax.axis_index("c")
    s = lax.axis_index("s")
    wid = s * 2 + c
    base = wid * BPW * L

    pltpu.sync_copy(idx_hbm.at[pl.ds(base, BPW * L)], idx_v)

    sems = (sem0, sem1)

    def start(b, slot):
        for (o, n) in CHUNKS:
            pltpu.async_copy(
                w_hbm.at[idx_v.at[pl.ds(b * L + o, n)]],
                rows_v.at[slot, pl.ds(o, n)],
                sems[slot],
            )

    def wait(slot):
        for (o, n) in CHUNKS:
            pltpu.make_async_copy(
                w_hbm.at[idx_v.at[pl.ds(o, n)]],
                rows_v.at[slot, pl.ds(o, n)],
                sems[slot],
            ).wait()

    start(0, 0)
    start(1, 1)

    def accum(slot, b):
        def inner(l, acc):
            return tuple(
                acc[q] + rows_v[slot, l, pl.ds(16 * q, 16)] for q in range(QV)
            )
        zero = jnp.zeros((16,), jnp.float32)
        acc = lax.fori_loop(0, L, inner, (zero,) * QV)
        scale = jnp.float32(1.0 / L)
        for q in range(QV):
            out_v[b // 2, pl.ds((b % 2) * EMB + 16 * q, 16)] = acc[q] * scale

    def outer(g, carry):
        for slot in range(2):
            b = g * 2 + slot
            wait(slot)
            accum(slot, b)
            nb = b + 2

            @pl.when(nb < BPW)
            def _():
                start(nb, slot)
        return carry

    lax.fori_loop(0, BPW // 2, outer, 0)

    pltpu.sync_copy(out_v, out_hbm.at[pl.ds(wid * (BPW // 2), BPW // 2)])


BK = 4096


def _repack_body(in_ref, out_ref):
    row = jax.lax.broadcasted_iota(jnp.int32, (EMB, EMB), 0)
    col = jax.lax.broadcasted_iota(jnp.int32, (EMB, EMB), 1)
    eye = (row == col).astype(jnp.float32)
    out_ref[:, :EMB] = jax.lax.dot_general(
        in_ref[...], eye, (((0,), (0,)), ((), ())),
        preferred_element_type=jnp.float32)


def _repack(wt):
    n = wt.shape[1]
    grid = (n + BK - 1) // BK
    return pl.pallas_call(
        _repack_body,
        grid=(grid,),
        in_specs=[pl.BlockSpec((EMB, BK), lambda j: (0, j))],
        out_specs=pl.BlockSpec((BK, WPAD), lambda j: (j, 0)),
        out_shape=jax.ShapeDtypeStruct((n, WPAD), jnp.float32),
    )(wt)


_mesh = plsc.VectorSubcoreMesh(core_axis_name="c", subcore_axis_name="s")

_sc_call = pl.kernel(
    _body,
    mesh=_mesh,
    out_type=jax.ShapeDtypeStruct((B // 2, WPAD), jnp.float32),
    scratch_types=[
        pltpu.VMEM((BPW * L,), jnp.int32),
        pltpu.VMEM((2, L, WPAD), jnp.float32),
        pltpu.VMEM((BPW // 2, WPAD), jnp.float32),
        pltpu.SemaphoreType.DMA,
        pltpu.SemaphoreType.DMA,
    ],
    compiler_params=pltpu.CompilerParams(use_tc_tiling_on_sc=True),
)


@jax.jit
def _run(x, w):
    w128 = _repack(w.T)
    out2 = _sc_call(x.reshape(B * L), w128)
    return out2.reshape(B, EMB)


def kernel(x, sizes, emb_weight):
    del sizes
    return _run(x, emb_weight)


# speedup vs baseline: 2.1663x; 1.1601x over previous
import functools

import jax
import jax.numpy as jnp
from jax import lax
from jax.experimental import pallas as pl
from jax.experimental.pallas import tpu as pltpu
from jax.experimental.pallas import tpu_sc as plsc

B = 4096
L = 200
EMB = 64
NW = 32
BPW = B // NW
CHUNKS = ((0, 128), (128, 72))
QV = EMB // 16
WPAD = 2 * EMB


def _body(idx_hbm, w_hbm, out_hbm, idx_v, rows_v, out_v, sem0, sem1):
    c = lax.axis_index("c")
    s = lax.axis_index("s")
    wid = s * 2 + c
    base = wid * BPW * L

    pltpu.sync_copy(idx_hbm.at[pl.ds(base, BPW * L)], idx_v)

    sems = (sem0, sem1)

    def start(b, slot):
        for (o, n) in CHUNKS:
            pltpu.async_copy(
                w_hbm.at[idx_v.at[pl.ds(b * L + o, n)]],
                rows_v.at[slot, pl.ds(o, n)],
                sems[slot],
            )

    def wait(slot):
        for (o, n) in CHUNKS:
            pltpu.make_async_copy(
                w_hbm.at[idx_v.at[pl.ds(o, n)]],
                rows_v.at[slot, pl.ds(o, n)],
                sems[slot],
            ).wait()

    start(0, 0)
    start(1, 1)

    def accum(slot, b):
        def inner(l, acc):
            return tuple(
                acc[q] + rows_v[slot, l, pl.ds(16 * q, 16)] for q in range(QV)
            )
        zero = jnp.zeros((16,), jnp.float32)
        acc = lax.fori_loop(0, L, inner, (zero,) * QV)
        scale = jnp.float32(1.0 / L)
        for q in range(QV):
            out_v[b // 2, pl.ds((b % 2) * EMB + 16 * q, 16)] = acc[q] * scale

    def outer(g, carry):
        for slot in range(2):
            b = g * 2 + slot
            wait(slot)
            accum(slot, b)
            nb = b + 2

            @pl.when(nb < BPW)
            def _():
                start(nb, slot)
        return carry

    lax.fori_loop(0, BPW // 2, outer, 0)

    pltpu.sync_copy(out_v, out_hbm.at[pl.ds(wid * (BPW // 2), BPW // 2)])


BK = 8192


def _repack_body(in_ref, out_ref):
    row = jax.lax.broadcasted_iota(jnp.int32, (EMB, EMB), 0)
    col = jax.lax.broadcasted_iota(jnp.int32, (EMB, EMB), 1)
    eye = (row == col).astype(jnp.float32)
    out_ref[:, :EMB] = jax.lax.dot_general(
        in_ref[...], eye, (((0,), (0,)), ((), ())),
        preferred_element_type=jnp.float32)


def _repack(wt):
    n = wt.shape[1]
    grid = (n + BK - 1) // BK
    return pl.pallas_call(
        _repack_body,
        grid=(grid,),
        in_specs=[pl.BlockSpec((EMB, BK), lambda j: (0, j))],
        out_specs=pl.BlockSpec((BK, WPAD), lambda j: (j, 0)),
        out_shape=jax.ShapeDtypeStruct((n, WPAD), jnp.float32),
    )(wt)


_mesh = plsc.VectorSubcoreMesh(core_axis_name="c", subcore_axis_name="s")

_sc_call = pl.kernel(
    _body,
    mesh=_mesh,
    out_type=jax.ShapeDtypeStruct((B // 2, WPAD), jnp.float32),
    scratch_types=[
        pltpu.VMEM((BPW * L,), jnp.int32),
        pltpu.VMEM((2, L, WPAD), jnp.float32),
        pltpu.VMEM((BPW // 2, WPAD), jnp.float32),
        pltpu.SemaphoreType.DMA,
        pltpu.SemaphoreType.DMA,
    ],
    compiler_params=pltpu.CompilerParams(use_tc_tiling_on_sc=True),
)


@jax.jit
def _run(x, w):
    w128 = _repack(w.T)
    out2 = _sc_call(x.reshape(B * L), w128)
    return out2.reshape(B, EMB)


def kernel(x, sizes, emb_weight):
    del sizes
    return _run(x, emb_weight)


# speedup vs baseline: 2.2798x; 1.0524x over previous
import functools

import jax
import jax.numpy as jnp
from jax import lax
from jax.experimental import pallas as pl
from jax.experimental.pallas import tpu as pltpu
from jax.experimental.pallas import tpu_sc as plsc

B = 4096
L = 200
EMB = 64
NW = 32
BPW = B // NW
CHUNKS = ((0, 128), (128, 72))
QV = EMB // 16
WPAD = 2 * EMB


def _body(idx_hbm, w_hbm, out_hbm, idx_v, rows_v, out_v, sem0, sem1):
    c = lax.axis_index("c")
    s = lax.axis_index("s")
    wid = s * 2 + c
    base = wid * BPW * L

    pltpu.sync_copy(idx_hbm.at[pl.ds(base, BPW * L)], idx_v)

    sems = (sem0, sem1)

    def start(b, slot):
        for (o, n) in CHUNKS:
            pltpu.async_copy(
                w_hbm.at[idx_v.at[pl.ds(b * L + o, n)]],
                rows_v.at[slot, pl.ds(o, n)],
                sems[slot],
            )

    def wait(slot):
        for (o, n) in CHUNKS:
            pltpu.make_async_copy(
                w_hbm.at[idx_v.at[pl.ds(o, n)]],
                rows_v.at[slot, pl.ds(o, n)],
                sems[slot],
            ).wait()

    start(0, 0)
    start(1, 1)

    def accum(slot, b):
        def inner(l, acc):
            return tuple(
                acc[q] + rows_v[slot, l, pl.ds(16 * q, 16)] for q in range(QV)
            )
        zero = jnp.zeros((16,), jnp.float32)
        acc = lax.fori_loop(0, L, inner, (zero,) * QV)
        scale = jnp.float32(1.0 / L)
        for q in range(QV):
            out_v[b // 2, pl.ds((b % 2) * EMB + 16 * q, 16)] = acc[q] * scale

    def outer(g, carry):
        for slot in range(2):
            b = g * 2 + slot
            wait(slot)
            accum(slot, b)
            nb = b + 2

            @pl.when(nb < BPW)
            def _():
                start(nb, slot)
        return carry

    lax.fori_loop(0, BPW // 2, outer, 0)

    pltpu.sync_copy(out_v, out_hbm.at[pl.ds(wid * (BPW // 2), BPW // 2)])


BK = 16384


def _repack_body(in_ref, out_ref):
    row = jax.lax.broadcasted_iota(jnp.int32, (EMB, EMB), 0)
    col = jax.lax.broadcasted_iota(jnp.int32, (EMB, EMB), 1)
    eye = (row == col).astype(jnp.float32)
    out_ref[:, :EMB] = jax.lax.dot_general(
        in_ref[...], eye, (((0,), (0,)), ((), ())),
        preferred_element_type=jnp.float32)


def _repack(wt):
    n = wt.shape[1]
    grid = (n + BK - 1) // BK
    return pl.pallas_call(
        _repack_body,
        grid=(grid,),
        in_specs=[pl.BlockSpec((EMB, BK), lambda j: (0, j))],
        out_specs=pl.BlockSpec((BK, WPAD), lambda j: (j, 0)),
        out_shape=jax.ShapeDtypeStruct((n, WPAD), jnp.float32),
    )(wt)


_mesh = plsc.VectorSubcoreMesh(core_axis_name="c", subcore_axis_name="s")

_sc_call = pl.kernel(
    _body,
    mesh=_mesh,
    out_type=jax.ShapeDtypeStruct((B // 2, WPAD), jnp.float32),
    scratch_types=[
        pltpu.VMEM((BPW * L,), jnp.int32),
        pltpu.VMEM((2, L, WPAD), jnp.float32),
        pltpu.VMEM((BPW // 2, WPAD), jnp.float32),
        pltpu.SemaphoreType.DMA,
        pltpu.SemaphoreType.DMA,
    ],
    compiler_params=pltpu.CompilerParams(use_tc_tiling_on_sc=True),
)


@jax.jit
def _run(x, w):
    w128 = _repack(w.T)
    out2 = _sc_call(x.reshape(B * L), w128)
    return out2.reshape(B, EMB)


def kernel(x, sizes, emb_weight):
    del sizes
    return _run(x, emb_weight)
